# baseline (speedup 1.0000x reference)
import functools
import math

import jax
import jax.numpy as jnp
from jax.experimental import pallas as pl
from jax.experimental.pallas import tpu as pltpu

_VOCAB = 100000
_PAD = 0
_SMOOTH = 0.1
_CONF = 1.0 - _SMOOTH
_FILL = _SMOOTH / (_VOCAB - 2)
_C = _CONF * math.log(_CONF) + _SMOOTH * math.log(_FILL)

_BLOCK_COLS = 4096


def _rowsum_kernel(y1_ref, y2_ref, acc_ref, z_ref, *, block_cols, vocab,
                   n_blocks, half):
    k = pl.program_id(0)

    @pl.when(k == 0)
    def _():
        z_ref[0:half, :] = y1_ref[:, 0:1]
        z_ref[half:, :] = y2_ref[:, 0:1]

    x1 = y1_ref[...]
    x2 = y2_ref[...]

    @pl.when(k < n_blocks - 1)
    def _():
        p1 = jnp.sum(x1, axis=1, keepdims=True)
        p2 = jnp.sum(x2, axis=1, keepdims=True)

        @pl.when(k == 0)
        def _():
            acc_ref[0:half, :] = p1
            acc_ref[half:, :] = p2

        @pl.when(k != 0)
        def _():
            acc_ref[0:half, :] += p1
            acc_ref[half:, :] += p2

    @pl.when(k == n_blocks - 1)
    def _():
        col = k * block_cols + jax.lax.broadcasted_iota(
            jnp.int32, (1, block_cols), 1)
        m = col < vocab
        acc_ref[0:half, :] += jnp.sum(jnp.where(m, x1, 0.0), axis=1,
                                      keepdims=True)
        acc_ref[half:, :] += jnp.sum(jnp.where(m, x2, 0.0), axis=1,
                                     keepdims=True)


def kernel(yhat, target):
    n, vocab = yhat.shape
    half = n // 2
    t = target.astype(jnp.int32)
    n_blocks = pl.cdiv(vocab, _BLOCK_COLS)
    rowsum, z = pl.pallas_call(
        functools.partial(_rowsum_kernel, block_cols=_BLOCK_COLS,
                          vocab=vocab, n_blocks=n_blocks, half=half),
        grid=(n_blocks,),
        in_specs=[
            pl.BlockSpec((half, _BLOCK_COLS), lambda k: (0, k)),
            pl.BlockSpec((half, _BLOCK_COLS), lambda k: (1, k)),
        ],
        out_specs=[
            pl.BlockSpec((n, 1), lambda k: (0, 0)),
            pl.BlockSpec((n, 1), lambda k: (0, 0)),
        ],
        out_shape=[
            jax.ShapeDtypeStruct((n, 1), jnp.float32),
            jax.ShapeDtypeStruct((n, 1), jnp.float32),
        ],
    )(yhat, yhat)

    g = jnp.take_along_axis(yhat, t[:, None], axis=1)
    valid = (t != _PAD).astype(jnp.float32)[:, None]
    per_row = _C - (_CONF - _FILL) * g - _FILL * (rowsum - z)
    return jnp.sum(per_row * valid)


# speedup vs baseline: 1.0060x; 1.0060x over previous
import functools
import math

import jax
import jax.numpy as jnp
from jax.experimental import pallas as pl
from jax.experimental.pallas import tpu as pltpu

_VOCAB = 100000
_PAD = 0
_SMOOTH = 0.1
_CONF = 1.0 - _SMOOTH
_FILL = _SMOOTH / (_VOCAB - 2)
_C = _CONF * math.log(_CONF) + _SMOOTH * math.log(_FILL)

_BLOCK_COLS = 4096
_N_STREAMS = 8


def _rowsum_kernel(*refs, block_cols, vocab, n_blocks, rows_per):
    y_refs = refs[:_N_STREAMS]
    acc_ref, z_ref = refs[_N_STREAMS:]
    k = pl.program_id(0)

    @pl.when(k == 0)
    def _():
        for s in range(_N_STREAMS):
            z_ref[s * rows_per:(s + 1) * rows_per, :] = y_refs[s][:, 0:1]

    @pl.when(k < n_blocks - 1)
    def _():
        for s in range(_N_STREAMS):
            p = jnp.sum(y_refs[s][...], axis=1, keepdims=True)
            lo = s * rows_per

            @pl.when(k == 0)
            def _(p=p, lo=lo):
                acc_ref[lo:lo + rows_per, :] = p

            @pl.when(k != 0)
            def _(p=p, lo=lo):
                acc_ref[lo:lo + rows_per, :] += p

    @pl.when(k == n_blocks - 1)
    def _():
        col = k * block_cols + jax.lax.broadcasted_iota(
            jnp.int32, (1, block_cols), 1)
        m = col < vocab
        for s in range(_N_STREAMS):
            p = jnp.sum(jnp.where(m, y_refs[s][...], 0.0), axis=1,
                        keepdims=True)
            lo = s * rows_per
            acc_ref[lo:lo + rows_per, :] += p


def kernel(yhat, target):
    n, vocab = yhat.shape
    rows_per = n // _N_STREAMS
    t = target.astype(jnp.int32)
    n_blocks = pl.cdiv(vocab, _BLOCK_COLS)
    rowsum, z = pl.pallas_call(
        functools.partial(_rowsum_kernel, block_cols=_BLOCK_COLS,
                          vocab=vocab, n_blocks=n_blocks, rows_per=rows_per),
        grid=(n_blocks,),
        in_specs=[
            pl.BlockSpec((rows_per, _BLOCK_COLS),
                         functools.partial(lambda s, k: (s, k), s))
            for s in range(_N_STREAMS)
        ],
        out_specs=[
            pl.BlockSpec((n, 1), lambda k: (0, 0)),
            pl.BlockSpec((n, 1), lambda k: (0, 0)),
        ],
        out_shape=[
            jax.ShapeDtypeStruct((n, 1), jnp.float32),
            jax.ShapeDtypeStruct((n, 1), jnp.float32),
        ],
    )(*([yhat] * _N_STREAMS))

    g = jnp.take_along_axis(yhat, t[:, None], axis=1)
    valid = (t != _PAD).astype(jnp.float32)[:, None]
    per_row = _C - (_CONF - _FILL) * g - _FILL * (rowsum - z)
    return jnp.sum(per_row * valid)
